# baseline (speedup 1.0000x reference)
import functools

import jax
import jax.numpy as jnp
from jax import lax
from jax.experimental import pallas as pl
from jax.experimental.pallas import tpu as pltpu
from jax.experimental.pallas import tpu_sc as plsc

B, C, N = 16, 3, 4096
LVL = 8
P = 64
LANES = 16
NCHUNK = N // LANES
NSROW = B * C
NREC = NSROW + P


def _iota():
    return lax.broadcasted_iota(jnp.int32, (LANES,), 0)


def _popcount32(v):
    v = v - (lax.shift_right_logical(v, 1) & 0x55555555)
    v = (v & 0x33333333) + (lax.shift_right_logical(v, 2) & 0x33333333)
    v = (v + lax.shift_right_logical(v, 4)) & 0x0F0F0F0F
    return lax.shift_right_logical(v * jnp.int32(0x01010101), 24)


def _set_bit(lo, hi, j):
    lo = lo | jnp.where(j < 32, lax.shift_left(jnp.int32(1), jnp.minimum(j, 31)), 0)
    hi = hi | jnp.where(j >= 32, lax.shift_left(jnp.int32(1), jnp.maximum(j - 32, 0)), 0)
    return lo, hi


def _make(interpret=False):
    mesh = plsc.VectorSubcoreMesh(core_axis_name="c", subcore_axis_name="s",
                                  num_cores=1)

    @functools.partial(
        pl.kernel,
        out_type=jax.ShapeDtypeStruct((B, P), jnp.float32),
        mesh=mesh,
        scratch_types=[
            pltpu.VMEM((N + 128,), jnp.int32),
            pltpu.VMEM((N,), jnp.float32),
            pltpu.VMEM((N + 128,), jnp.int32),
            pltpu.VMEM((N,), jnp.int32),
            pltpu.VMEM((128,), jnp.int32),
            pltpu.VMEM((C, LANES), jnp.float32),
            pltpu.VMEM((LANES,), jnp.int32),
            pltpu.VMEM((LANES,), jnp.float32),
            pltpu.VMEM_SHARED((NREC * LANES,), jnp.int32),
            pltpu.VMEM((NREC * LANES,), jnp.int32),
        ],
        compiler_params=pltpu.CompilerParams(needs_layout_passes=False),
        interpret=interpret,
    )
    def ccp(x_hbm, curve_hbm, levels_hbm, pmap_hbm, out_hbm,
            curve_v, xrow_v, prow_v, qrow_v, pres_v, lev_v,
            stage_i, stage_f, recs_sh, recs_v):
        cid = lax.axis_index("c")
        sid = lax.axis_index("s")
        iota = _iota()
        ones16 = jnp.full((LANES,), 1, jnp.int32)
        zeros16 = jnp.zeros((LANES,), jnp.int32)

        pltpu.sync_copy(curve_hbm, curve_v.at[pl.ds(0, N)])
        curve_v[pl.ds(N, LANES)] = zeros16
        pltpu.sync_copy(levels_hbm, lev_v)

        def clear_presence():
            for j in range(P // LANES):
                pres_v[pl.ds(j * LANES, LANES)] = zeros16

        def emit_record(row, first, last):
            parts = []
            for j in range(P // LANES):
                pj = pres_v[pl.ds(j * LANES, LANES)]
                parts.append(jnp.sum(lax.shift_left(pj, iota)))
            lo = parts[0] | lax.shift_left(parts[1], 16)
            hi = parts[2] | lax.shift_left(parts[3], 16)
            rec = (jnp.where(iota == 0, lo, 0)
                   | jnp.where(iota == 1, hi, 0)
                   | jnp.where(iota == 2, first, 0)
                   | jnp.where(iota == 3, last, 0))
            stage_i[...] = rec
            pltpu.sync_copy(stage_i, recs_sh.at[pl.ds(row * LANES, LANES)])

        def do_s_row(r):
            b = r // C
            ch = r % C
            pltpu.sync_copy(x_hbm.at[b, ch], xrow_v)
            lev_row = lev_v[ch]
            levs = [lev_row[l] for l in range(LVL)]

            @plsc.parallel_loop(0, NCHUNK, unroll=4)
            def _quant(i):
                v = xrow_v[pl.ds(i * LANES, LANES)]
                best = jnp.abs(v - levs[0])
                q = zeros16
                for l in range(1, LVL):
                    d = jnp.abs(v - levs[l])
                    take = d < best
                    q = jnp.where(take, jnp.int32(l), q)
                    best = jnp.where(take, d, best)
                qrow_v[pl.ds(i * LANES, LANES)] = q

            clear_presence()

            @plsc.parallel_loop(0, NCHUNK, unroll=4)
            def _codes(i):
                base = i * LANES
                i0 = curve_v[pl.ds(base, LANES)]
                i1 = curve_v[pl.ds(base + 1, LANES)]
                q0 = plsc.load_gather(qrow_v, [i0])
                q1 = plsc.load_gather(qrow_v, [i1])
                code = q0 * LVL + q1
                msk = (base + iota) < (N - 1)
                plsc.store_scatter(pres_v, [code], ones16, mask=msk)

            q_head = plsc.load_gather(qrow_v, [curve_v[pl.ds(0, LANES)]])
            q_tail = plsc.load_gather(qrow_v, [curve_v[pl.ds(N - LANES, LANES)]])
            emit_record(r, q_head[0], q_tail[LANES - 1])

        def do_p_row(p):
            pltpu.sync_copy(pmap_hbm.at[p], prow_v.at[pl.ds(0, N)])
            prow_v[pl.ds(N, LANES)] = zeros16
            clear_presence()

            @plsc.parallel_loop(0, NCHUNK, unroll=4)
            def _codes(i):
                base = i * LANES
                a = prow_v[pl.ds(base, LANES)]
                b2 = prow_v[pl.ds(base + 1, LANES)]
                code = a * LVL + b2
                msk = (base + iota) < (N - 1)
                plsc.store_scatter(pres_v, [code], ones16, mask=msk)

            p_head = prow_v[pl.ds(0, LANES)]
            p_tail = prow_v[pl.ds(N - LANES, LANES)]
            emit_record(NSROW + p, p_head[0], p_tail[LANES - 1])

        for k in range(3):
            do_s_row(sid + 16 * k)
        for k in range(4):
            do_p_row(sid * 4 + k)

        plsc.subcore_barrier()

        pltpu.sync_copy(recs_sh, recs_v)

        b = sid
        r0 = b * C
        if True:
            rec0 = recs_v[pl.ds(r0 * LANES, LANES)]
            rec1 = recs_v[pl.ds((r0 + 1) * LANES, LANES)]
            rec2 = recs_v[pl.ds((r0 + 2) * LANES, LANES)]
            lo = rec0[0] | rec1[0] | rec2[0]
            hi = rec0[1] | rec1[1] | rec2[1]
            j1 = rec0[3] * LVL + rec1[2]
            j2 = rec1[3] * LVL + rec2[2]
            lo, hi = _set_bit(lo, hi, j1)
            lo, hi = _set_bit(lo, hi, j2)
            s_last = rec2[3]
            lo_v = jnp.broadcast_to(lo, (LANES,))
            hi_v = jnp.broadcast_to(hi, (LANES,))
            cs = (_popcount32(lo_v) + _popcount32(hi_v)).astype(jnp.float32)

        for pv in range(4):
            rows = (NSROW + pv * LANES + iota) * LANES
            p_lo = plsc.load_gather(recs_v, [rows])
            p_hi = plsc.load_gather(recs_v, [rows + 1])
            p_first = plsc.load_gather(recs_v, [rows + 2])

            u_lo = p_lo | lo_v
            u_hi = p_hi | hi_v
            pc = _popcount32(u_lo) + _popcount32(u_hi)
            j = s_last * LVL + p_first
            bit = jnp.where(
                j < 32,
                lax.shift_right_logical(u_lo, jnp.minimum(j, 31)) & 1,
                lax.shift_right_logical(u_hi, jnp.maximum(j - 32, 0)) & 1,
            )
            csp = (pc + 1 - bit).astype(jnp.float32)
            cp = (_popcount32(p_lo) + _popcount32(p_hi)).astype(jnp.float32)
            ncd = (csp - jnp.minimum(cs, cp)) / jnp.maximum(cs, cp)
            stage_f[...] = ncd
            pltpu.sync_copy(stage_f, out_hbm.at[b, pl.ds(pv * LANES, LANES)])

    return ccp


@functools.cache
def _get_ccp():
    return _make()


def kernel(x, curve, levels, pmap):
    xf = x.reshape(B, C, N)
    pf = pmap.reshape(-1, pmap.shape[-1]).astype(jnp.int32)
    lev_pad = jnp.concatenate(
        [levels.astype(jnp.float32),
         jnp.zeros((C, LANES - LVL), jnp.float32)], axis=1)
    return _get_ccp()(xf, curve.astype(jnp.int32), lev_pad, pf)


# speedup vs baseline: 1.0785x; 1.0785x over previous
import functools

import jax
import jax.numpy as jnp
from jax import lax
from jax.experimental import pallas as pl
from jax.experimental.pallas import tpu as pltpu
from jax.experimental.pallas import tpu_sc as plsc

B, C, N = 16, 3, 4096
LVL = 8
P = 64
LANES = 16
NCHUNK = N // LANES
NSROW = (B // 2) * C
NREC = NSROW + P


def _iota():
    return lax.broadcasted_iota(jnp.int32, (LANES,), 0)


def _popcount32(v):
    v = v - (lax.shift_right_logical(v, 1) & 0x55555555)
    v = (v & 0x33333333) + (lax.shift_right_logical(v, 2) & 0x33333333)
    v = (v + lax.shift_right_logical(v, 4)) & 0x0F0F0F0F
    return lax.shift_right_logical(v * jnp.int32(0x01010101), 24)


def _set_bit(lo, hi, j):
    lo = lo | jnp.where(j < 32, lax.shift_left(jnp.int32(1), jnp.minimum(j, 31)), 0)
    hi = hi | jnp.where(j >= 32, lax.shift_left(jnp.int32(1), jnp.maximum(j - 32, 0)), 0)
    return lo, hi


def _make(interpret=False):
    mesh = plsc.VectorSubcoreMesh(core_axis_name="c", subcore_axis_name="s")

    @functools.partial(
        pl.kernel,
        out_type=jax.ShapeDtypeStruct((B, P), jnp.float32),
        mesh=mesh,
        scratch_types=[
            pltpu.VMEM((N + 128,), jnp.int32),
            pltpu.VMEM((N,), jnp.float32),
            pltpu.VMEM((N + 128,), jnp.int32),
            pltpu.VMEM((N,), jnp.int32),
            pltpu.VMEM((128,), jnp.int32),
            pltpu.VMEM((C, LANES), jnp.float32),
            pltpu.VMEM((LANES,), jnp.int32),
            pltpu.VMEM((LANES,), jnp.float32),
            pltpu.VMEM_SHARED((NREC * LANES,), jnp.int32),
            pltpu.VMEM((NREC * LANES,), jnp.int32),
        ],
        compiler_params=pltpu.CompilerParams(needs_layout_passes=False),
        interpret=interpret,
    )
    def ccp(x_hbm, curve_hbm, levels_hbm, pmap_hbm, out_hbm,
            curve_v, xrow_v, prow_v, qrow_v, pres_v, lev_v,
            stage_i, stage_f, recs_sh, recs_v):
        cid = lax.axis_index("c")
        sid = lax.axis_index("s")
        iota = _iota()
        ones16 = jnp.full((LANES,), 1, jnp.int32)
        zeros16 = jnp.zeros((LANES,), jnp.int32)

        pltpu.sync_copy(curve_hbm, curve_v.at[pl.ds(0, N)])
        curve_v[pl.ds(N, LANES)] = zeros16
        pltpu.sync_copy(levels_hbm, lev_v)

        def clear_presence():
            for j in range(P // LANES):
                pres_v[pl.ds(j * LANES, LANES)] = zeros16

        def emit_record(row, first, last):
            parts = []
            for j in range(P // LANES):
                pj = pres_v[pl.ds(j * LANES, LANES)]
                parts.append(jnp.sum(lax.shift_left(pj, iota)))
            lo = parts[0] | lax.shift_left(parts[1], 16)
            hi = parts[2] | lax.shift_left(parts[3], 16)
            rec = (jnp.where(iota == 0, lo, 0)
                   | jnp.where(iota == 1, hi, 0)
                   | jnp.where(iota == 2, first, 0)
                   | jnp.where(iota == 3, last, 0))
            stage_i[...] = rec
            pltpu.sync_copy(stage_i, recs_sh.at[pl.ds(row * LANES, LANES)])

        def do_s_row(r):
            b_local = r // C
            ch = r % C
            b = cid * (B // 2) + b_local
            pltpu.sync_copy(x_hbm.at[b, ch], xrow_v)
            lev_row = lev_v[ch]
            levs = [lev_row[l] for l in range(LVL)]

            @plsc.parallel_loop(0, NCHUNK, unroll=4)
            def _quant(i):
                v = xrow_v[pl.ds(i * LANES, LANES)]
                best = jnp.abs(v - levs[0])
                q = zeros16
                for l in range(1, LVL):
                    d = jnp.abs(v - levs[l])
                    take = d < best
                    q = jnp.where(take, jnp.int32(l), q)
                    best = jnp.where(take, d, best)
                qrow_v[pl.ds(i * LANES, LANES)] = q

            clear_presence()

            @plsc.parallel_loop(0, NCHUNK, unroll=4)
            def _codes(i):
                base = i * LANES
                i0 = curve_v[pl.ds(base, LANES)]
                i1 = curve_v[pl.ds(base + 1, LANES)]
                q0 = plsc.load_gather(qrow_v, [i0])
                q1 = plsc.load_gather(qrow_v, [i1])
                code = q0 * LVL + q1
                msk = (base + iota) < (N - 1)
                plsc.store_scatter(pres_v, [code], ones16, mask=msk)

            q_head = plsc.load_gather(qrow_v, [curve_v[pl.ds(0, LANES)]])
            q_tail = plsc.load_gather(qrow_v, [curve_v[pl.ds(N - LANES, LANES)]])
            emit_record(r, q_head[0], q_tail[LANES - 1])

        def do_p_row(p):
            pltpu.sync_copy(pmap_hbm.at[p], prow_v.at[pl.ds(0, N)])
            prow_v[pl.ds(N, LANES)] = zeros16
            clear_presence()

            @plsc.parallel_loop(0, NCHUNK, unroll=4)
            def _codes(i):
                base = i * LANES
                a = prow_v[pl.ds(base, LANES)]
                b2 = prow_v[pl.ds(base + 1, LANES)]
                code = a * LVL + b2
                msk = (base + iota) < (N - 1)
                plsc.store_scatter(pres_v, [code], ones16, mask=msk)

            p_head = prow_v[pl.ds(0, LANES)]
            p_tail = prow_v[pl.ds(N - LANES, LANES)]
            emit_record(NSROW + p, p_head[0], p_tail[LANES - 1])

        def s_loop(k, carry):
            r = sid + 16 * k

            @pl.when(r < NSROW)
            def _():
                do_s_row(r)

            return carry

        lax.fori_loop(0, 2, s_loop, None)

        heavy = sid < 8
        p_base = jnp.where(heavy, sid, 8 + (sid - 8) * 7)
        p_cnt = jnp.where(heavy, 1, 7)

        def p_loop(k, carry):
            do_p_row(p_base + k)
            return carry

        lax.fori_loop(0, p_cnt, p_loop, None)

        plsc.subcore_barrier()

        pltpu.sync_copy(recs_sh, recs_v)

        def combine(t, carry):
            u = sid * 2 + t
            b_local = u // 4
            pv = u % 4
            b = cid * (B // 2) + b_local
            r0 = b_local * C
            rec0 = recs_v[pl.ds(r0 * LANES, LANES)]
            rec1 = recs_v[pl.ds((r0 + 1) * LANES, LANES)]
            rec2 = recs_v[pl.ds((r0 + 2) * LANES, LANES)]
            lo = rec0[0] | rec1[0] | rec2[0]
            hi = rec0[1] | rec1[1] | rec2[1]
            j1 = rec0[3] * LVL + rec1[2]
            j2 = rec1[3] * LVL + rec2[2]
            lo, hi = _set_bit(lo, hi, j1)
            lo, hi = _set_bit(lo, hi, j2)
            s_last = rec2[3]

            rows = (NSROW + pv * LANES + iota) * LANES
            p_lo = plsc.load_gather(recs_v, [rows])
            p_hi = plsc.load_gather(recs_v, [rows + 1])
            p_first = plsc.load_gather(recs_v, [rows + 2])

            lo_v = jnp.broadcast_to(lo, (LANES,))
            hi_v = jnp.broadcast_to(hi, (LANES,))
            u_lo = p_lo | lo_v
            u_hi = p_hi | hi_v
            pc = _popcount32(u_lo) + _popcount32(u_hi)
            j = s_last * LVL + p_first
            bit = jnp.where(
                j < 32,
                lax.shift_right_logical(u_lo, jnp.minimum(j, 31)) & 1,
                lax.shift_right_logical(u_hi, jnp.maximum(j - 32, 0)) & 1,
            )
            csp = (pc + 1 - bit).astype(jnp.float32)
            cs = (_popcount32(lo_v) + _popcount32(hi_v)).astype(jnp.float32)
            cp = (_popcount32(p_lo) + _popcount32(p_hi)).astype(jnp.float32)
            ncd = (csp - jnp.minimum(cs, cp)) / jnp.maximum(cs, cp)
            stage_f[...] = ncd
            pltpu.sync_copy(stage_f, out_hbm.at[b, pl.ds(pv * LANES, LANES)])
            return carry

        lax.fori_loop(0, 2, combine, None)

    return ccp


@functools.cache
def _get_ccp():
    return _make()


def kernel(x, curve, levels, pmap):
    xf = x.reshape(B, C, N)
    pf = pmap.reshape(-1, pmap.shape[-1]).astype(jnp.int32)
    lev_pad = jnp.concatenate(
        [levels.astype(jnp.float32),
         jnp.zeros((C, LANES - LVL), jnp.float32)], axis=1)
    return _get_ccp()(xf, curve.astype(jnp.int32), lev_pad, pf)
